# baseline (speedup 1.0000x reference)
import functools

import jax
import jax.numpy as jnp
from jax import lax
from jax.experimental import pallas as pl
from jax.experimental.pallas import tpu as pltpu
from jax.experimental.pallas import tpu_sc as plsc

B = 1024
L = 512
D = 64
GENE_V = 100000
EXPR_V = 1000

NC = 2
NS = 16
NW = NC * NS
TOK = B * L
TPW = TOK // NW
CH = 256
NCHUNK = TPW // CH
SUB = 128
CPW = B // NW


def _premix_gene_body(g_ref, w_ref, o_ref):
    w1 = w_ref[...][:, :D]
    h = lax.dot_general(g_ref[...], w1, (((1,), (1,)), ((), ())),
                        preferred_element_type=jnp.float32)
    o_ref[...] = jnp.concatenate([h, jnp.zeros_like(h)], axis=1)


def _premix_small_body(e_ref, w_ref, b_ref, c_ref, l_ref, eo_ref, co_ref,
                       lo_ref):
    w2 = w_ref[...][:, D:]
    h = lax.dot_general(e_ref[...], w2, (((1,), (1,)), ((), ())),
                        preferred_element_type=jnp.float32) + b_ref[...]
    z = jnp.zeros_like(h)
    eo_ref[...] = jnp.concatenate([z, h], axis=1)
    co_ref[...] = jnp.concatenate([c_ref[...], z], axis=1)
    lo_ref[...] = jnp.concatenate([z, l_ref[...]], axis=1)


def _ln(x, gamma, beta):
    mean = jnp.mean(x, axis=-1, keepdims=True)
    xc = x - mean
    var = jnp.mean(xc * xc, axis=-1, keepdims=True)
    return xc * lax.rsqrt(var + 1e-5) * gamma + beta


def _finish_body(s_ref, pos_ref, extra_ref, gamma_ref, beta_ref, o_ref):
    x = jnp.maximum(s_ref[...], 0.0)
    g = gamma_ref[...]
    b = beta_ref[...]
    pos = pos_ref[...]
    ya = _ln(x[..., :D], g, b) + pos[..., :D]
    yb = _ln(x[..., D:], g, b) + pos[..., D:]
    y = jnp.concatenate([ya, yb], axis=-1)
    epad = jnp.concatenate([extra_ref[...], jnp.zeros_like(extra_ref[...])],
                           axis=-1)
    lmask = (lax.broadcasted_iota(jnp.int32, (1, L // 2, 1), 1) == 0)
    y = y + jnp.where(lmask, 1.0, 0.0) * epad[:, None, :]
    o_ref[...] = y


def _sc_gather_add(gm_hbm, em_hbm, gi_hbm, ei_hbm, ct_hbm, lt_hbm, ci_hbm,
                   li_hbm, s_hbm, extra_hbm,
                   gidx_v, eidx_v, rows_g, rows_e, rows_o, cidx_v, lidx_v,
                   crow, lrow, orow, sem):
    wid = lax.axis_index("s") * NC + lax.axis_index("c")

    cb = wid * CPW
    pltpu.sync_copy(ci_hbm.at[pl.ds(cb, CPW)], cidx_v)
    pltpu.sync_copy(li_hbm.at[pl.ds(cb, CPW)], lidx_v)
    pltpu.async_copy(ct_hbm.at[cidx_v], crow, sem).wait()
    pltpu.async_copy(lt_hbm.at[lidx_v], lrow, sem).wait()

    @pl.loop(0, CPW)
    def _(r):
        for i in range(D // 16):
            sl = pl.ds(i * 16, 16)
            sh = pl.ds(D + i * 16, 16)
            orow[r, sl] = crow[r, sl] + lrow[r, sh]

    pltpu.sync_copy(orow, extra_hbm.at[pl.ds(cb, CPW)])

    base0 = wid * TPW

    base2_0 = wid * (TPW // 2)

    @pl.loop(0, NCHUNK)
    def _(ch):
        base = base0 + ch * CH
        base2 = base2_0 + ch * (CH // 2)
        pltpu.sync_copy(gi_hbm.at[pl.ds(base, CH)], gidx_v)
        pltpu.sync_copy(ei_hbm.at[pl.ds(base, CH)], eidx_v)
        copies = []
        for j in range(CH // SUB):
            sl = pl.ds(j * SUB, SUB)
            copies.append(pltpu.async_copy(
                gm_hbm.at[gidx_v.at[sl]], rows_g.at[sl], sem))
            copies.append(pltpu.async_copy(
                em_hbm.at[eidx_v.at[sl]], rows_e.at[sl], sem))
        for c in copies:
            c.wait()

        @pl.loop(0, CH // 2)
        def _(r2):
            for t in range(2):
                r = 2 * r2 + t
                for i in range(D // 16):
                    sl = pl.ds(i * 16, 16)
                    sh = pl.ds(D + i * 16, 16)
                    so = pl.ds(t * D + i * 16, 16)
                    rows_o[r2, so] = rows_g[r, sl] + rows_e[r, sh]

        pltpu.sync_copy(rows_o, s_hbm.at[pl.ds(base2, CH // 2)])


@functools.cache
def _sc_gather_call():
    return functools.partial(
        pl.kernel,
        out_type=(jax.ShapeDtypeStruct((TOK // 2, 2 * D), jnp.float32),
                  jax.ShapeDtypeStruct((B, D), jnp.float32)),
        mesh=plsc.VectorSubcoreMesh(core_axis_name="c", subcore_axis_name="s"),
        scratch_types=[
            pltpu.VMEM((CH,), jnp.int32),
            pltpu.VMEM((CH,), jnp.int32),
            pltpu.VMEM((CH, 2 * D), jnp.float32),
            pltpu.VMEM((CH, 2 * D), jnp.float32),
            pltpu.VMEM((CH // 2, 2 * D), jnp.float32),
            pltpu.VMEM((CPW,), jnp.int32),
            pltpu.VMEM((CPW,), jnp.int32),
            pltpu.VMEM((CPW, 2 * D), jnp.float32),
            pltpu.VMEM((CPW, 2 * D), jnp.float32),
            pltpu.VMEM((CPW, D), jnp.float32),
            pltpu.SemaphoreType.DMA,
        ],
    )(_sc_gather_add)


GBLK = 4000
RB = 8


def kernel(gene_ids, expression_tokens, condition_tokens, library_size,
           gene_table, expr_table, cond_table, lib_table, pos_table,
           W_mix, b_mix, ln_gamma, ln_beta):
    gi = jnp.asarray(gene_ids, jnp.int32).reshape(TOK)
    ei = jnp.asarray(expression_tokens, jnp.int32).reshape(TOK)
    ci = jnp.asarray(condition_tokens, jnp.int32)
    li = jnp.asarray(library_size, jnp.int32)

    gene_mixed = pl.pallas_call(
        _premix_gene_body,
        grid=(GENE_V // GBLK,),
        in_specs=[pl.BlockSpec((GBLK, D), lambda i: (i, 0)),
                  pl.BlockSpec((D, 2 * D), lambda i: (0, 0))],
        out_specs=pl.BlockSpec((GBLK, 2 * D), lambda i: (i, 0)),
        out_shape=jax.ShapeDtypeStruct((GENE_V, 2 * D), jnp.float32),
    )(gene_table, W_mix)

    expr_mixed, cond_wide, lib_wide = pl.pallas_call(
        _premix_small_body,
        grid=(1,),
        in_specs=[pl.BlockSpec((EXPR_V, D), lambda i: (0, 0)),
                  pl.BlockSpec((D, 2 * D), lambda i: (0, 0)),
                  pl.BlockSpec((1, D), lambda i: (0, 0)),
                  pl.BlockSpec((EXPR_V, D), lambda i: (0, 0)),
                  pl.BlockSpec((EXPR_V, D), lambda i: (0, 0))],
        out_specs=[pl.BlockSpec((EXPR_V, 2 * D), lambda i: (0, 0)),
                   pl.BlockSpec((EXPR_V, 2 * D), lambda i: (0, 0)),
                   pl.BlockSpec((EXPR_V, 2 * D), lambda i: (0, 0))],
        out_shape=[jax.ShapeDtypeStruct((EXPR_V, 2 * D), jnp.float32),
                   jax.ShapeDtypeStruct((EXPR_V, 2 * D), jnp.float32),
                   jax.ShapeDtypeStruct((EXPR_V, 2 * D), jnp.float32)],
    )(expr_table, W_mix, b_mix.reshape(1, D), cond_table, lib_table)

    s, extra = _sc_gather_call()(gene_mixed, expr_mixed, gi, ei,
                                 cond_wide, lib_wide, ci, li)

    out2 = pl.pallas_call(
        _finish_body,
        grid=(B // RB,),
        in_specs=[pl.BlockSpec((RB, L // 2, 2 * D), lambda i: (i, 0, 0)),
                  pl.BlockSpec((L // 2, 2 * D), lambda i: (0, 0)),
                  pl.BlockSpec((RB, D), lambda i: (i, 0)),
                  pl.BlockSpec((1, D), lambda i: (0, 0)),
                  pl.BlockSpec((1, D), lambda i: (0, 0))],
        out_specs=pl.BlockSpec((RB, L // 2, 2 * D), lambda i: (i, 0, 0)),
        out_shape=jax.ShapeDtypeStruct((B, L // 2, 2 * D), jnp.float32),
    )(s.reshape(B, L // 2, 2 * D), pos_table.reshape(L // 2, 2 * D), extra,
      ln_gamma.reshape(1, D), ln_beta.reshape(1, D))

    return out2.reshape(B, L, D)


# speedup vs baseline: 1.4466x; 1.4466x over previous
import functools

import jax
import jax.numpy as jnp
from jax import lax
from jax.experimental import pallas as pl
from jax.experimental.pallas import tpu as pltpu
from jax.experimental.pallas import tpu_sc as plsc

B = 1024
L = 512
D = 64
GENE_V = 100000
EXPR_V = 1000

NC = 2
NS = 16
NW = NC * NS
TOK = B * L
TPW = TOK // NW
CH = 128
NCHUNK = TPW // CH
CPW = B // NW


def _premix_gene_body(g_ref, w_ref, o_ref):
    w1 = w_ref[...][:, :D]
    h = lax.dot_general(g_ref[...], w1, (((1,), (1,)), ((), ())),
                        preferred_element_type=jnp.float32)
    o_ref[...] = jnp.concatenate([h, jnp.zeros_like(h)], axis=1)


def _premix_small_body(e_ref, w_ref, b_ref, c_ref, l_ref, eo_ref, co_ref,
                       lo_ref):
    w2 = w_ref[...][:, D:]
    h = lax.dot_general(e_ref[...], w2, (((1,), (1,)), ((), ())),
                        preferred_element_type=jnp.float32) + b_ref[...]
    z = jnp.zeros_like(h)
    eo_ref[...] = jnp.concatenate([z, h], axis=1)
    co_ref[...] = jnp.concatenate([c_ref[...], z], axis=1)
    lo_ref[...] = jnp.concatenate([z, l_ref[...]], axis=1)


def _ln(x, gamma, beta):
    mean = jnp.mean(x, axis=-1, keepdims=True)
    xc = x - mean
    var = jnp.mean(xc * xc, axis=-1, keepdims=True)
    return xc * lax.rsqrt(var + 1e-5) * gamma + beta


def _finish_body(s_ref, pos_ref, extra_ref, gamma_ref, beta_ref, o_ref):
    x = jnp.maximum(s_ref[...], 0.0)
    y = _ln(x, gamma_ref[...], beta_ref[...]) + pos_ref[...]
    lmask = (lax.broadcasted_iota(jnp.int32, (1, L, 1), 1) == 0)
    y = y + jnp.where(lmask, 1.0, 0.0) * extra_ref[...][:, None, :]
    o_ref[...] = y


def _sc_gather_add(gm_hbm, em_hbm, gi_hbm, ei_hbm, ct_hbm, lt_hbm, ci_hbm,
                   li_hbm, s_hbm, extra_hbm,
                   gidx_all, eidx_all, rows_g0, rows_g1, rows_e0, rows_e1,
                   rows_ob, cidx_v, lidx_v, crow, lrow, orow,
                   sem_g0, sem_g1, sem_e0, sem_e1, sem):
    wid = lax.axis_index("s") * NC + lax.axis_index("c")
    rows_g = (rows_g0, rows_g1)
    rows_e = (rows_e0, rows_e1)
    rows_o = (rows_ob, rows_ob)
    sem_g = (sem_g0, sem_g1)
    sem_e = (sem_e0, sem_e1)

    cb = wid * CPW
    pltpu.sync_copy(ci_hbm.at[pl.ds(cb, CPW)], cidx_v)
    pltpu.sync_copy(li_hbm.at[pl.ds(cb, CPW)], lidx_v)
    pltpu.async_copy(ct_hbm.at[cidx_v], crow, sem).wait()
    pltpu.async_copy(lt_hbm.at[lidx_v], lrow, sem).wait()

    @pl.loop(0, CPW)
    def _(r):
        for i in range(D // 16):
            sl = pl.ds(i * 16, 16)
            sh = pl.ds(D + i * 16, 16)
            orow[r, sl] = crow[r, sl] + lrow[r, sh]

    pltpu.sync_copy(orow, extra_hbm.at[pl.ds(cb, CPW)])

    base0 = wid * TPW
    pltpu.sync_copy(gi_hbm.at[pl.ds(base0, TPW)], gidx_all)
    pltpu.sync_copy(ei_hbm.at[pl.ds(base0, TPW)], eidx_all)

    def fire(ch, b):
        sl = pl.ds(ch * CH, CH)
        pltpu.async_copy(gm_hbm.at[gidx_all.at[sl]], rows_g[b], sem_g[b])
        pltpu.async_copy(em_hbm.at[eidx_all.at[sl]], rows_e[b], sem_e[b])

    def drain(ch, b):
        sl = pl.ds(ch * CH, CH)
        pltpu.make_async_copy(gm_hbm.at[gidx_all.at[sl]], rows_g[b],
                              sem_g[b]).wait()
        pltpu.make_async_copy(em_hbm.at[eidx_all.at[sl]], rows_e[b],
                              sem_e[b]).wait()

    def process(ch, b):
        rg, re, ro = rows_g[b], rows_e[b], rows_o[b]

        @pl.loop(0, CH)
        def _(r):
            for i in range(D // 16):
                sl = pl.ds(i * 16, 16)
                sh = pl.ds(D + i * 16, 16)
                ro[r, sl] = rg[r, sl] + re[r, sh]

        pltpu.sync_copy(ro, s_hbm.at[pl.ds(base0 + ch * CH, CH)])

    fire(0, 0)
    fire(1, 1)

    @pl.loop(0, NCHUNK, step=2)
    def _(ch):
        drain(ch, 0)

        @pl.when(ch + 2 < NCHUNK)
        def _():
            fire(ch + 2, 0)

        process(ch, 0)
        drain(ch + 1, 1)

        @pl.when(ch + 3 < NCHUNK)
        def _():
            fire(ch + 3, 1)

        process(ch + 1, 1)


@functools.cache
def _sc_gather_call():
    return functools.partial(
        pl.kernel,
        out_type=(jax.ShapeDtypeStruct((TOK, D), jnp.float32),
                  jax.ShapeDtypeStruct((B, D), jnp.float32)),
        mesh=plsc.VectorSubcoreMesh(core_axis_name="c", subcore_axis_name="s"),
        scratch_types=[
            pltpu.VMEM((TPW,), jnp.int32),
            pltpu.VMEM((TPW,), jnp.int32),
            pltpu.VMEM((CH, 2 * D), jnp.float32),
            pltpu.VMEM((CH, 2 * D), jnp.float32),
            pltpu.VMEM((CH, 2 * D), jnp.float32),
            pltpu.VMEM((CH, 2 * D), jnp.float32),
            pltpu.VMEM((CH, D), jnp.float32),
            pltpu.VMEM((CPW,), jnp.int32),
            pltpu.VMEM((CPW,), jnp.int32),
            pltpu.VMEM((CPW, 2 * D), jnp.float32),
            pltpu.VMEM((CPW, 2 * D), jnp.float32),
            pltpu.VMEM((CPW, D), jnp.float32),
            pltpu.SemaphoreType.DMA,
            pltpu.SemaphoreType.DMA,
            pltpu.SemaphoreType.DMA,
            pltpu.SemaphoreType.DMA,
            pltpu.SemaphoreType.DMA,
        ],
    )(_sc_gather_add)


GBLK = 4000
RB = 8


def kernel(gene_ids, expression_tokens, condition_tokens, library_size,
           gene_table, expr_table, cond_table, lib_table, pos_table,
           W_mix, b_mix, ln_gamma, ln_beta):
    gi = jnp.asarray(gene_ids, jnp.int32).reshape(TOK)
    ei = jnp.asarray(expression_tokens, jnp.int32).reshape(TOK)
    ci = jnp.asarray(condition_tokens, jnp.int32)
    li = jnp.asarray(library_size, jnp.int32)

    gene_mixed = pl.pallas_call(
        _premix_gene_body,
        grid=(GENE_V // GBLK,),
        in_specs=[pl.BlockSpec((GBLK, D), lambda i: (i, 0)),
                  pl.BlockSpec((D, 2 * D), lambda i: (0, 0))],
        out_specs=pl.BlockSpec((GBLK, 2 * D), lambda i: (i, 0)),
        out_shape=jax.ShapeDtypeStruct((GENE_V, 2 * D), jnp.float32),
    )(gene_table, W_mix)

    expr_mixed, cond_wide, lib_wide = pl.pallas_call(
        _premix_small_body,
        grid=(1,),
        in_specs=[pl.BlockSpec((EXPR_V, D), lambda i: (0, 0)),
                  pl.BlockSpec((D, 2 * D), lambda i: (0, 0)),
                  pl.BlockSpec((1, D), lambda i: (0, 0)),
                  pl.BlockSpec((EXPR_V, D), lambda i: (0, 0)),
                  pl.BlockSpec((EXPR_V, D), lambda i: (0, 0))],
        out_specs=[pl.BlockSpec((EXPR_V, 2 * D), lambda i: (0, 0)),
                   pl.BlockSpec((EXPR_V, 2 * D), lambda i: (0, 0)),
                   pl.BlockSpec((EXPR_V, 2 * D), lambda i: (0, 0))],
        out_shape=[jax.ShapeDtypeStruct((EXPR_V, 2 * D), jnp.float32),
                   jax.ShapeDtypeStruct((EXPR_V, 2 * D), jnp.float32),
                   jax.ShapeDtypeStruct((EXPR_V, 2 * D), jnp.float32)],
    )(expr_table, W_mix, b_mix.reshape(1, D), cond_table, lib_table)

    s, extra = _sc_gather_call()(gene_mixed, expr_mixed, gi, ei,
                                 cond_wide, lib_wide, ci, li)

    out = pl.pallas_call(
        _finish_body,
        grid=(B // RB,),
        in_specs=[pl.BlockSpec((RB, L, D), lambda i: (i, 0, 0)),
                  pl.BlockSpec((L, D), lambda i: (0, 0)),
                  pl.BlockSpec((RB, D), lambda i: (i, 0)),
                  pl.BlockSpec((1, D), lambda i: (0, 0)),
                  pl.BlockSpec((1, D), lambda i: (0, 0))],
        out_specs=pl.BlockSpec((RB, L, D), lambda i: (i, 0, 0)),
        out_shape=jax.ShapeDtypeStruct((B, L, D), jnp.float32),
    )(s.reshape(B, L, D), pos_table, extra,
      ln_gamma.reshape(1, D), ln_beta.reshape(1, D))

    return out


# speedup vs baseline: 1.9445x; 1.3442x over previous
import dataclasses
import functools

import jax
import jax.numpy as jnp
from jax import lax
from jax.experimental import pallas as pl
from jax.experimental.pallas import tpu as pltpu
from jax.experimental.pallas import tpu_sc as plsc

B = 1024
L = 512
D = 64
GENE_V = 100000
EXPR_V = 1000

NC = 2
NS = 16
NW = NC * NS
TOK = B * L
TPW = TOK // NW
CH = 64
NCHUNK = TPW // CH
CPB = L // CH
CPW = B // NW


def _premix_gene_body(g_ref, w_ref, o_ref):
    w1 = w_ref[...][:, :D]
    h = lax.dot_general(g_ref[...], w1, (((1,), (1,)), ((), ())),
                        preferred_element_type=jnp.float32)
    o_ref[...] = jnp.concatenate([h, jnp.zeros_like(h)], axis=1)


def _premix_small_body(e_ref, w_ref, b_ref, c_ref, l_ref, p_ref, beta_ref,
                       eo_ref, co_ref, lo_ref, po_ref):
    w2 = w_ref[...][:, D:]
    h = lax.dot_general(e_ref[...], w2, (((1,), (1,)), ((), ())),
                        preferred_element_type=jnp.float32) + b_ref[...]
    z = jnp.zeros_like(h)
    eo_ref[...] = jnp.concatenate([z, h], axis=1)
    co_ref[...] = jnp.concatenate([c_ref[...], z], axis=1)
    lo_ref[...] = jnp.concatenate([z, l_ref[...]], axis=1)
    po_ref[...] = p_ref[...] + beta_ref[...]


def _rsqrt_vec(v):
    i = lax.bitcast_convert_type(v, jnp.int32)
    i = jnp.int32(0x5F3759DF) - lax.shift_right_arithmetic(i, jnp.int32(1))
    y = lax.bitcast_convert_type(i, jnp.float32)
    for _ in range(3):
        y = y * (1.5 - 0.5 * v * y * y)
    return y


def _sc_gather_ln(gm_hbm, em_hbm, gi_hbm, ei_hbm, ct_hbm, lt_hbm, ci_hbm,
                  li_hbm, posb_hbm, gamma_hbm, out_hbm,
                  gidx_all, eidx_all, rows_g0, rows_g1, rows_e0, rows_e1,
                  rows_ob, posb_v, gamma_v, cidx_v, lidx_v, orow,
                  sem_g0, sem_g1, sem_e0, sem_e1, sem):
    wid = lax.axis_index("s") * NC + lax.axis_index("c")
    rows_g = (rows_g0, rows_g1)
    rows_e = (rows_e0, rows_e1)
    sem_g = (sem_g0, sem_g1)
    sem_e = (sem_e0, sem_e1)

    pltpu.sync_copy(posb_hbm, posb_v)
    pltpu.sync_copy(gamma_hbm, gamma_v)

    crow = rows_g0.at[pl.ds(0, CPW)]
    lrow = rows_e0.at[pl.ds(0, CPW)]
    cb = wid * CPW
    pltpu.sync_copy(ci_hbm.at[pl.ds(cb, CPW)], cidx_v)
    pltpu.sync_copy(li_hbm.at[pl.ds(cb, CPW)], lidx_v)
    pltpu.async_copy(ct_hbm.at[cidx_v], crow, sem).wait()
    pltpu.async_copy(lt_hbm.at[lidx_v], lrow, sem).wait()

    @pl.loop(0, CPW)
    def _(r):
        for i in range(D // 16):
            sl = pl.ds(i * 16, 16)
            sh = pl.ds(D + i * 16, 16)
            orow[r, sl] = rows_g0[r, sl] + rows_e0[r, sh]

    base0 = wid * TPW
    gvs = [gamma_v[pl.ds(i * 16, 16)] for i in range(D // 16)]

    NCH = NCHUNK // 2
    HTOK = TPW // 2

    def fire(c, b):
        sl = pl.ds(c * CH, CH)
        pltpu.async_copy(gm_hbm.at[gidx_all.at[sl]], rows_g[b], sem_g[b])
        pltpu.async_copy(em_hbm.at[eidx_all.at[sl]], rows_e[b], sem_e[b])

    def drain(c, b):
        sl = pl.ds(c * CH, CH)
        pltpu.make_async_copy(gm_hbm.at[gidx_all.at[sl]], rows_g[b],
                              sem_g[b]).wait()
        pltpu.make_async_copy(em_hbm.at[eidx_all.at[sl]], rows_e[b],
                              sem_e[b]).wait()

    def process(ch, b):
        rg, re, ro = rows_g[b], rows_e[b], rows_ob
        po = (ch % CPB) * CH

        @pl.loop(0, CH)
        def _(r):
            vs = []
            for i in range(D // 16):
                sl = pl.ds(i * 16, 16)
                sh = pl.ds(D + i * 16, 16)
                vs.append(jnp.maximum(rg[r, sl] + re[r, sh], 0.0))
            s1 = (vs[0] + vs[1]) + (vs[2] + vs[3])
            s2 = (vs[0] * vs[0] + vs[1] * vs[1]) + \
                 (vs[2] * vs[2] + vs[3] * vs[3])
            mean = jnp.sum(s1) * (1.0 / D)
            var = jnp.sum(s2) * (1.0 / D) - mean * mean + 1e-5
            rs = _rsqrt_vec(jnp.broadcast_to(var, (16,)))
            vmean = jnp.broadcast_to(mean, (16,))
            p = po + r
            for i in range(D // 16):
                sl = pl.ds(i * 16, 16)
                ro[r, sl] = (vs[i] - vmean) * (rs * gvs[i]) + posb_v[p, sl]

        @pl.when(ch % CPB == 0)
        def _():
            bl = ch // CPB
            for i in range(D // 16):
                sl = pl.ds(i * 16, 16)
                ro[0, sl] = ro[0, sl] + orow[bl, sl]

        pltpu.sync_copy(ro, out_hbm.at[pl.ds(base0 + ch * CH, CH)])

    for h in range(2):
        hb = h * HTOK
        pltpu.sync_copy(gi_hbm.at[pl.ds(base0 + hb, HTOK)], gidx_all)
        pltpu.sync_copy(ei_hbm.at[pl.ds(base0 + hb, HTOK)], eidx_all)
        fire(0, 0)
        fire(1, 1)

        @pl.loop(0, NCH, step=2)
        def _(c):
            drain(c, 0)

            @pl.when(c + 2 < NCH)
            def _():
                fire(c + 2, 0)

            process(h * NCH + c, 0)
            drain(c + 1, 1)

            @pl.when(c + 3 < NCH)
            def _():
                fire(c + 3, 1)

            process(h * NCH + c + 1, 1)


@functools.cache
def _sc_gather_call():
    cp = pltpu.CompilerParams()
    if "needs_layout_passes" in pltpu.CompilerParams.__dataclass_fields__:
        cp = dataclasses.replace(cp, needs_layout_passes=False)
    return functools.partial(
        pl.kernel,
        out_type=jax.ShapeDtypeStruct((TOK, D), jnp.float32),
        compiler_params=cp,
        mesh=plsc.VectorSubcoreMesh(core_axis_name="c", subcore_axis_name="s"),
        scratch_types=[
            pltpu.VMEM((TPW // 2,), jnp.int32),
            pltpu.VMEM((TPW // 2,), jnp.int32),
            pltpu.VMEM((CH, 2 * D), jnp.float32),
            pltpu.VMEM((CH, 2 * D), jnp.float32),
            pltpu.VMEM((CH, 2 * D), jnp.float32),
            pltpu.VMEM((CH, 2 * D), jnp.float32),
            pltpu.VMEM((CH, D), jnp.float32),
            pltpu.VMEM((L, D), jnp.float32),
            pltpu.VMEM((D,), jnp.float32),
            pltpu.VMEM((CPW,), jnp.int32),
            pltpu.VMEM((CPW,), jnp.int32),
            pltpu.VMEM((CPW, D), jnp.float32),
            pltpu.SemaphoreType.DMA,
            pltpu.SemaphoreType.DMA,
            pltpu.SemaphoreType.DMA,
            pltpu.SemaphoreType.DMA,
            pltpu.SemaphoreType.DMA,
        ],
    )(_sc_gather_ln)


GBLK = 4000


def kernel(gene_ids, expression_tokens, condition_tokens, library_size,
           gene_table, expr_table, cond_table, lib_table, pos_table,
           W_mix, b_mix, ln_gamma, ln_beta):
    gi = jnp.asarray(gene_ids, jnp.int32).reshape(TOK)
    ei = jnp.asarray(expression_tokens, jnp.int32).reshape(TOK)
    ci = jnp.asarray(condition_tokens, jnp.int32)
    li = jnp.asarray(library_size, jnp.int32)

    gene_mixed = pl.pallas_call(
        _premix_gene_body,
        grid=(GENE_V // GBLK,),
        in_specs=[pl.BlockSpec((GBLK, D), lambda i: (i, 0)),
                  pl.BlockSpec((D, 2 * D), lambda i: (0, 0))],
        out_specs=pl.BlockSpec((GBLK, 2 * D), lambda i: (i, 0)),
        out_shape=jax.ShapeDtypeStruct((GENE_V, 2 * D), jnp.float32),
    )(gene_table, W_mix)

    expr_mixed, cond_wide, lib_wide, posb = pl.pallas_call(
        _premix_small_body,
        grid=(1,),
        in_specs=[pl.BlockSpec((EXPR_V, D), lambda i: (0, 0)),
                  pl.BlockSpec((D, 2 * D), lambda i: (0, 0)),
                  pl.BlockSpec((1, D), lambda i: (0, 0)),
                  pl.BlockSpec((EXPR_V, D), lambda i: (0, 0)),
                  pl.BlockSpec((EXPR_V, D), lambda i: (0, 0)),
                  pl.BlockSpec((L, D), lambda i: (0, 0)),
                  pl.BlockSpec((1, D), lambda i: (0, 0))],
        out_specs=[pl.BlockSpec((EXPR_V, 2 * D), lambda i: (0, 0)),
                   pl.BlockSpec((EXPR_V, 2 * D), lambda i: (0, 0)),
                   pl.BlockSpec((EXPR_V, 2 * D), lambda i: (0, 0)),
                   pl.BlockSpec((L, D), lambda i: (0, 0))],
        out_shape=[jax.ShapeDtypeStruct((EXPR_V, 2 * D), jnp.float32),
                   jax.ShapeDtypeStruct((EXPR_V, 2 * D), jnp.float32),
                   jax.ShapeDtypeStruct((EXPR_V, 2 * D), jnp.float32),
                   jax.ShapeDtypeStruct((L, D), jnp.float32)],
    )(expr_table, W_mix, b_mix.reshape(1, D), cond_table, lib_table,
      pos_table, ln_beta.reshape(1, D))

    out = _sc_gather_call()(gene_mixed, expr_mixed, gi, ei,
                            cond_wide, lib_wide, ci, li, posb, ln_gamma)

    return out.reshape(B, L, D)


# speedup vs baseline: 1.9452x; 1.0003x over previous
import dataclasses
import functools

import jax
import jax.numpy as jnp
from jax import lax
from jax.experimental import pallas as pl
from jax.experimental.pallas import tpu as pltpu
from jax.experimental.pallas import tpu_sc as plsc

B = 1024
L = 512
D = 64
GENE_V = 100000
EXPR_V = 1000

NC = 2
NS = 16
NW = NC * NS
TOK = B * L
TPW = TOK // NW
CH = 64
NCHUNK = TPW // CH
CPB = L // CH
CPW = B // NW


def _premix_gene_body(g_ref, w_ref, o_ref):
    w1 = w_ref[...][:, :D]
    h = lax.dot_general(g_ref[...], w1, (((1,), (1,)), ((), ())),
                        preferred_element_type=jnp.float32)
    o_ref[...] = jnp.concatenate([h, jnp.zeros_like(h)], axis=1)


def _premix_small_body(e_ref, w_ref, b_ref, c_ref, l_ref, p_ref, beta_ref,
                       eo_ref, co_ref, lo_ref, po_ref):
    w2 = w_ref[...][:, D:]
    h = lax.dot_general(e_ref[...], w2, (((1,), (1,)), ((), ())),
                        preferred_element_type=jnp.float32) + b_ref[...]
    z = jnp.zeros_like(h)
    eo_ref[...] = jnp.concatenate([z, h], axis=1)
    co_ref[...] = jnp.concatenate([c_ref[...], z], axis=1)
    lo_ref[...] = jnp.concatenate([z, l_ref[...]], axis=1)
    po_ref[...] = p_ref[...] + beta_ref[...]


def _rsqrt_vec(v):
    i = lax.bitcast_convert_type(v, jnp.int32)
    i = jnp.int32(0x5F3759DF) - lax.shift_right_arithmetic(i, jnp.int32(1))
    y = lax.bitcast_convert_type(i, jnp.float32)
    for _ in range(3):
        y = y * (1.5 - 0.5 * v * y * y)
    return y


def _sc_gather_ln(gm_hbm, em_hbm, gi_hbm, ei_hbm, ct_hbm, lt_hbm, ci_hbm,
                  li_hbm, posb_hbm, gamma_hbm, out_hbm,
                  gidx_all, eidx_all, rows_g0, rows_g1, rows_e0, rows_e1,
                  rows_ob, posb_v, gamma_v, cidx_v, lidx_v, orow,
                  sem_g0, sem_g1, sem_e0, sem_e1, sem):
    wid = lax.axis_index("s") * NC + lax.axis_index("c")
    rows_g = (rows_g0, rows_g1)
    rows_e = (rows_e0, rows_e1)
    sem_g = (sem_g0, sem_g1)
    sem_e = (sem_e0, sem_e1)

    pltpu.sync_copy(posb_hbm, posb_v)
    pltpu.sync_copy(gamma_hbm, gamma_v)

    crow = rows_g0.at[pl.ds(0, CPW)]
    lrow = rows_e0.at[pl.ds(0, CPW)]
    cb = wid * CPW
    pltpu.sync_copy(ci_hbm.at[pl.ds(cb, CPW)], cidx_v)
    pltpu.sync_copy(li_hbm.at[pl.ds(cb, CPW)], lidx_v)
    pltpu.async_copy(ct_hbm.at[cidx_v], crow, sem).wait()
    pltpu.async_copy(lt_hbm.at[lidx_v], lrow, sem).wait()

    @pl.loop(0, CPW)
    def _(r):
        for i in range(D // 16):
            sl = pl.ds(i * 16, 16)
            sh = pl.ds(D + i * 16, 16)
            orow[r, sl] = rows_g0[r, sl] + rows_e0[r, sh]

    base0 = wid * TPW
    gvs = [gamma_v[pl.ds(i * 16, 16)] for i in range(D // 16)]

    NCH = NCHUNK // 2
    HTOK = TPW // 2

    def fire(c, b):
        sl = pl.ds(c * CH, CH)
        pltpu.async_copy(gm_hbm.at[gidx_all.at[sl]], rows_g[b], sem_g[b])
        pltpu.async_copy(em_hbm.at[eidx_all.at[sl]], rows_e[b], sem_e[b])

    def drain(c, b):
        sl = pl.ds(c * CH, CH)
        pltpu.make_async_copy(gm_hbm.at[gidx_all.at[sl]], rows_g[b],
                              sem_g[b]).wait()
        pltpu.make_async_copy(em_hbm.at[eidx_all.at[sl]], rows_e[b],
                              sem_e[b]).wait()

    def process(ch, b):
        rg, re, ro = rows_g[b], rows_e[b], rows_ob
        po = (ch % CPB) * CH

        @pl.loop(0, CH)
        def _(r):
            vs = []
            for i in range(D // 16):
                sl = pl.ds(i * 16, 16)
                sh = pl.ds(D + i * 16, 16)
                vs.append(jnp.maximum(rg[r, sl] + re[r, sh], 0.0))
            s1 = (vs[0] + vs[1]) + (vs[2] + vs[3])
            s2 = (vs[0] * vs[0] + vs[1] * vs[1]) + \
                 (vs[2] * vs[2] + vs[3] * vs[3])
            mean = jnp.sum(s1) * (1.0 / D)
            var = jnp.sum(s2) * (1.0 / D) - mean * mean + 1e-5
            rs = _rsqrt_vec(jnp.broadcast_to(var, (16,)))
            vmean = jnp.broadcast_to(mean, (16,))
            p = po + r
            for i in range(D // 16):
                sl = pl.ds(i * 16, 16)
                ro[r, sl] = (vs[i] - vmean) * (rs * gvs[i]) + posb_v[p, sl]

        @pl.when(ch % CPB == 0)
        def _():
            bl = ch // CPB
            for i in range(D // 16):
                sl = pl.ds(i * 16, 16)
                ro[0, sl] = ro[0, sl] + orow[bl, sl]

        pltpu.sync_copy(ro, out_hbm.at[pl.ds(base0 + ch * CH, CH)])

    for h in range(2):
        hb = h * HTOK
        pltpu.sync_copy(gi_hbm.at[pl.ds(base0 + hb, HTOK)], gidx_all)
        pltpu.sync_copy(ei_hbm.at[pl.ds(base0 + hb, HTOK)], eidx_all)
        fire(0, 0)
        fire(1, 1)

        @pl.loop(0, NCH, step=2)
        def _(c):
            drain(c, 0)
            process(h * NCH + c, 0)

            @pl.when(c + 2 < NCH)
            def _():
                fire(c + 2, 0)

            drain(c + 1, 1)
            process(h * NCH + c + 1, 1)

            @pl.when(c + 3 < NCH)
            def _():
                fire(c + 3, 1)


@functools.cache
def _sc_gather_call():
    cp = pltpu.CompilerParams()
    if "needs_layout_passes" in pltpu.CompilerParams.__dataclass_fields__:
        cp = dataclasses.replace(cp, needs_layout_passes=False)
    return functools.partial(
        pl.kernel,
        out_type=jax.ShapeDtypeStruct((TOK, D), jnp.float32),
        compiler_params=cp,
        mesh=plsc.VectorSubcoreMesh(core_axis_name="c", subcore_axis_name="s"),
        scratch_types=[
            pltpu.VMEM((TPW // 2,), jnp.int32),
            pltpu.VMEM((TPW // 2,), jnp.int32),
            pltpu.VMEM((CH, 2 * D), jnp.float32),
            pltpu.VMEM((CH, 2 * D), jnp.float32),
            pltpu.VMEM((CH, 2 * D), jnp.float32),
            pltpu.VMEM((CH, 2 * D), jnp.float32),
            pltpu.VMEM((CH, D), jnp.float32),
            pltpu.VMEM((L, D), jnp.float32),
            pltpu.VMEM((D,), jnp.float32),
            pltpu.VMEM((CPW,), jnp.int32),
            pltpu.VMEM((CPW,), jnp.int32),
            pltpu.VMEM((CPW, D), jnp.float32),
            pltpu.SemaphoreType.DMA,
            pltpu.SemaphoreType.DMA,
            pltpu.SemaphoreType.DMA,
            pltpu.SemaphoreType.DMA,
            pltpu.SemaphoreType.DMA,
        ],
    )(_sc_gather_ln)


GBLK = 4000


def kernel(gene_ids, expression_tokens, condition_tokens, library_size,
           gene_table, expr_table, cond_table, lib_table, pos_table,
           W_mix, b_mix, ln_gamma, ln_beta):
    gi = jnp.asarray(gene_ids, jnp.int32).reshape(TOK)
    ei = jnp.asarray(expression_tokens, jnp.int32).reshape(TOK)
    ci = jnp.asarray(condition_tokens, jnp.int32)
    li = jnp.asarray(library_size, jnp.int32)

    gene_mixed = pl.pallas_call(
        _premix_gene_body,
        grid=(GENE_V // GBLK,),
        in_specs=[pl.BlockSpec((GBLK, D), lambda i: (i, 0)),
                  pl.BlockSpec((D, 2 * D), lambda i: (0, 0))],
        out_specs=pl.BlockSpec((GBLK, 2 * D), lambda i: (i, 0)),
        out_shape=jax.ShapeDtypeStruct((GENE_V, 2 * D), jnp.float32),
    )(gene_table, W_mix)

    expr_mixed, cond_wide, lib_wide, posb = pl.pallas_call(
        _premix_small_body,
        grid=(1,),
        in_specs=[pl.BlockSpec((EXPR_V, D), lambda i: (0, 0)),
                  pl.BlockSpec((D, 2 * D), lambda i: (0, 0)),
                  pl.BlockSpec((1, D), lambda i: (0, 0)),
                  pl.BlockSpec((EXPR_V, D), lambda i: (0, 0)),
                  pl.BlockSpec((EXPR_V, D), lambda i: (0, 0)),
                  pl.BlockSpec((L, D), lambda i: (0, 0)),
                  pl.BlockSpec((1, D), lambda i: (0, 0))],
        out_specs=[pl.BlockSpec((EXPR_V, 2 * D), lambda i: (0, 0)),
                   pl.BlockSpec((EXPR_V, 2 * D), lambda i: (0, 0)),
                   pl.BlockSpec((EXPR_V, 2 * D), lambda i: (0, 0)),
                   pl.BlockSpec((L, D), lambda i: (0, 0))],
        out_shape=[jax.ShapeDtypeStruct((EXPR_V, 2 * D), jnp.float32),
                   jax.ShapeDtypeStruct((EXPR_V, 2 * D), jnp.float32),
                   jax.ShapeDtypeStruct((EXPR_V, 2 * D), jnp.float32),
                   jax.ShapeDtypeStruct((L, D), jnp.float32)],
    )(expr_table, W_mix, b_mix.reshape(1, D), cond_table, lib_table,
      pos_table, ln_beta.reshape(1, D))

    out = _sc_gather_call()(gene_mixed, expr_mixed, gi, ei,
                            cond_wide, lib_wide, ci, li, posb, ln_gamma)

    return out.reshape(B, L, D)


# speedup vs baseline: 2.1130x; 1.0863x over previous
import dataclasses
import functools

import jax
import jax.numpy as jnp
from jax import lax
from jax.experimental import pallas as pl
from jax.experimental.pallas import tpu as pltpu
from jax.experimental.pallas import tpu_sc as plsc

B = 1024
L = 512
D = 64
GENE_V = 100000
EXPR_V = 1000

NC = 2
NS = 16
NW = NC * NS
TOK = B * L
TPW = TOK // NW
CH = 64
NCHUNK = TPW // CH
CPB = L // CH
CPW = B // NW


def _premix_gene_body(g_ref, w_ref, o_ref):
    w1 = w_ref[...][:, :D]
    h = lax.dot_general(g_ref[...], w1, (((1,), (1,)), ((), ())),
                        preferred_element_type=jnp.float32)
    o_ref[...] = jnp.concatenate([h, jnp.zeros_like(h)], axis=1)


def _premix_small_body(e_ref, w_ref, b_ref, c_ref, l_ref, p_ref, beta_ref,
                       eo_ref, co_ref, lo_ref, po_ref):
    w2 = w_ref[...][:, D:]
    h = lax.dot_general(e_ref[...], w2, (((1,), (1,)), ((), ())),
                        preferred_element_type=jnp.float32) + b_ref[...]
    z = jnp.zeros_like(h)
    eo_ref[...] = jnp.concatenate([z, h], axis=1)
    co_ref[...] = jnp.concatenate([c_ref[...], z], axis=1)
    lo_ref[...] = jnp.concatenate([z, l_ref[...]], axis=1)
    po_ref[...] = p_ref[...] + beta_ref[...]


def _rsqrt_vec(v):
    i = lax.bitcast_convert_type(v, jnp.int32)
    i = jnp.int32(0x5F3759DF) - lax.shift_right_arithmetic(i, jnp.int32(1))
    y = lax.bitcast_convert_type(i, jnp.float32)
    for _ in range(2):
        y = y * (1.5 - 0.5 * v * y * y)
    return y


def _sc_gather_ln(gm_hbm, em_hbm, gi_hbm, ei_hbm, ct_hbm, lt_hbm, ci_hbm,
                  li_hbm, posb_hbm, gamma_hbm, out_hbm,
                  gidx_all, eidx_all, rows_g0, rows_g1, rows_e0, rows_e1,
                  rows_o0, rows_o1, posb_v, gamma_v, cidx_v, lidx_v, orow,
                  sem_g0, sem_g1, sem_e0, sem_e1, sem_w0, sem_w1, sem):
    wid = lax.axis_index("s") * NC + lax.axis_index("c")
    rows_g = (rows_g0, rows_g1)
    rows_e = (rows_e0, rows_e1)
    rows_o = (rows_o0, rows_o1)
    sem_w = (sem_w0, sem_w1)
    sem_g = (sem_g0, sem_g1)
    sem_e = (sem_e0, sem_e1)

    pltpu.sync_copy(posb_hbm, posb_v)
    pltpu.sync_copy(gamma_hbm, gamma_v)

    crow = rows_g0.at[pl.ds(0, CPW)]
    lrow = rows_e0.at[pl.ds(0, CPW)]
    cb = wid * CPW
    pltpu.sync_copy(ci_hbm.at[pl.ds(cb, CPW)], cidx_v)
    pltpu.sync_copy(li_hbm.at[pl.ds(cb, CPW)], lidx_v)
    pltpu.async_copy(ct_hbm.at[cidx_v], crow, sem).wait()
    pltpu.async_copy(lt_hbm.at[lidx_v], lrow, sem).wait()

    @pl.loop(0, CPW)
    def _(r):
        for i in range(D // 16):
            sl = pl.ds(i * 16, 16)
            sh = pl.ds(D + i * 16, 16)
            orow[r, sl] = rows_g0[r, sl] + rows_e0[r, sh]

    base0 = wid * TPW
    gvs = [gamma_v[pl.ds(i * 16, 16)] for i in range(D // 16)]

    NCH = NCHUNK // 4
    HTOK = TPW // 4

    def fire(c, b):
        sl = pl.ds(c * CH, CH)
        pltpu.async_copy(gm_hbm.at[gidx_all.at[sl]], rows_g[b], sem_g[b])
        pltpu.async_copy(em_hbm.at[eidx_all.at[sl]], rows_e[b], sem_e[b])

    def drain_w(b):
        pltpu.make_async_copy(rows_o[b], out_hbm.at[pl.ds(base0, CH)],
                              sem_w[b]).wait()

    def drain(c, b):
        sl = pl.ds(c * CH, CH)
        pltpu.make_async_copy(gm_hbm.at[gidx_all.at[sl]], rows_g[b],
                              sem_g[b]).wait()
        pltpu.make_async_copy(em_hbm.at[eidx_all.at[sl]], rows_e[b],
                              sem_e[b]).wait()

    def process(ch, b):
        rg, re, ro = rows_g[b], rows_e[b], rows_o[b]
        po = (ch % CPB) * CH

        @pl.loop(0, CH)
        def _(r):
            vs = []
            for i in range(D // 16):
                sl = pl.ds(i * 16, 16)
                sh = pl.ds(D + i * 16, 16)
                vs.append(jnp.maximum(rg[r, sl] + re[r, sh], 0.0))
            s1 = (vs[0] + vs[1]) + (vs[2] + vs[3])
            s2 = (vs[0] * vs[0] + vs[1] * vs[1]) + \
                 (vs[2] * vs[2] + vs[3] * vs[3])
            mean = jnp.sum(s1) * (1.0 / D)
            var = jnp.sum(s2) * (1.0 / D) - mean * mean + 1e-5
            rs = _rsqrt_vec(jnp.broadcast_to(var, (16,)))
            vmean = jnp.broadcast_to(mean, (16,))
            p = po + r
            for i in range(D // 16):
                sl = pl.ds(i * 16, 16)
                ro[r, sl] = (vs[i] - vmean) * (rs * gvs[i]) + posb_v[p, sl]

        @pl.when(ch % CPB == 0)
        def _():
            bl = ch // CPB
            for i in range(D // 16):
                sl = pl.ds(i * 16, 16)
                ro[0, sl] = ro[0, sl] + orow[bl, sl]

        pltpu.async_copy(ro, out_hbm.at[pl.ds(base0 + ch * CH, CH)],
                         sem_w[b])

    for h in range(4):
        hb = h * HTOK
        pltpu.sync_copy(gi_hbm.at[pl.ds(base0 + hb, HTOK)], gidx_all)
        pltpu.sync_copy(ei_hbm.at[pl.ds(base0 + hb, HTOK)], eidx_all)
        fire(0, 0)
        fire(1, 1)

        @pl.loop(0, NCH, step=2)
        def _(c):
            drain(c, 0)

            @pl.when(c >= 2)
            def _():
                drain_w(0)

            process(h * NCH + c, 0)

            @pl.when(c + 2 < NCH)
            def _():
                fire(c + 2, 0)

            drain(c + 1, 1)

            @pl.when(c >= 2)
            def _():
                drain_w(1)

            process(h * NCH + c + 1, 1)

            @pl.when(c + 3 < NCH)
            def _():
                fire(c + 3, 1)

        drain_w(0)
        drain_w(1)


@functools.cache
def _sc_gather_call():
    cp = pltpu.CompilerParams()
    if "needs_layout_passes" in pltpu.CompilerParams.__dataclass_fields__:
        cp = dataclasses.replace(cp, needs_layout_passes=False)
    return functools.partial(
        pl.kernel,
        out_type=jax.ShapeDtypeStruct((TOK, D), jnp.float32),
        compiler_params=cp,
        mesh=plsc.VectorSubcoreMesh(core_axis_name="c", subcore_axis_name="s"),
        scratch_types=[
            pltpu.VMEM((TPW // 4,), jnp.int32),
            pltpu.VMEM((TPW // 4,), jnp.int32),
            pltpu.VMEM((CH, 2 * D), jnp.float32),
            pltpu.VMEM((CH, 2 * D), jnp.float32),
            pltpu.VMEM((CH, 2 * D), jnp.float32),
            pltpu.VMEM((CH, 2 * D), jnp.float32),
            pltpu.VMEM((CH, D), jnp.float32),
            pltpu.VMEM((CH, D), jnp.float32),
            pltpu.VMEM((L, D), jnp.float32),
            pltpu.VMEM((D,), jnp.float32),
            pltpu.VMEM((CPW,), jnp.int32),
            pltpu.VMEM((CPW,), jnp.int32),
            pltpu.VMEM((CPW, D), jnp.float32),
            pltpu.SemaphoreType.DMA,
            pltpu.SemaphoreType.DMA,
            pltpu.SemaphoreType.DMA,
            pltpu.SemaphoreType.DMA,
            pltpu.SemaphoreType.DMA,
            pltpu.SemaphoreType.DMA,
            pltpu.SemaphoreType.DMA,
        ],
    )(_sc_gather_ln)


GBLK = 4000


def kernel(gene_ids, expression_tokens, condition_tokens, library_size,
           gene_table, expr_table, cond_table, lib_table, pos_table,
           W_mix, b_mix, ln_gamma, ln_beta):
    gi = jnp.asarray(gene_ids, jnp.int32).reshape(TOK)
    ei = jnp.asarray(expression_tokens, jnp.int32).reshape(TOK)
    ci = jnp.asarray(condition_tokens, jnp.int32)
    li = jnp.asarray(library_size, jnp.int32)

    gene_mixed = pl.pallas_call(
        _premix_gene_body,
        grid=(GENE_V // GBLK,),
        in_specs=[pl.BlockSpec((GBLK, D), lambda i: (i, 0)),
                  pl.BlockSpec((D, 2 * D), lambda i: (0, 0))],
        out_specs=pl.BlockSpec((GBLK, 2 * D), lambda i: (i, 0)),
        out_shape=jax.ShapeDtypeStruct((GENE_V, 2 * D), jnp.float32),
    )(gene_table, W_mix)

    expr_mixed, cond_wide, lib_wide, posb = pl.pallas_call(
        _premix_small_body,
        grid=(1,),
        in_specs=[pl.BlockSpec((EXPR_V, D), lambda i: (0, 0)),
                  pl.BlockSpec((D, 2 * D), lambda i: (0, 0)),
                  pl.BlockSpec((1, D), lambda i: (0, 0)),
                  pl.BlockSpec((EXPR_V, D), lambda i: (0, 0)),
                  pl.BlockSpec((EXPR_V, D), lambda i: (0, 0)),
                  pl.BlockSpec((L, D), lambda i: (0, 0)),
                  pl.BlockSpec((1, D), lambda i: (0, 0))],
        out_specs=[pl.BlockSpec((EXPR_V, 2 * D), lambda i: (0, 0)),
                   pl.BlockSpec((EXPR_V, 2 * D), lambda i: (0, 0)),
                   pl.BlockSpec((EXPR_V, 2 * D), lambda i: (0, 0)),
                   pl.BlockSpec((L, D), lambda i: (0, 0))],
        out_shape=[jax.ShapeDtypeStruct((EXPR_V, 2 * D), jnp.float32),
                   jax.ShapeDtypeStruct((EXPR_V, 2 * D), jnp.float32),
                   jax.ShapeDtypeStruct((EXPR_V, 2 * D), jnp.float32),
                   jax.ShapeDtypeStruct((L, D), jnp.float32)],
    )(expr_table, W_mix, b_mix.reshape(1, D), cond_table, lib_table,
      pos_table, ln_beta.reshape(1, D))

    out = _sc_gather_call()(gene_mixed, expr_mixed, gi, ei,
                            cond_wide, lib_wide, ci, li, posb, ln_gamma)

    return out.reshape(B, L, D)
